# baseline (speedup 1.0000x reference)
import functools

import jax
import jax.numpy as jnp
from jax import lax
from jax.experimental import pallas as pl
from jax.experimental.pallas import tpu as pltpu
from jax.experimental.pallas import tpu_sc as plsc

NC = 2
NS = 16
NW = NC * NS
LANES = 16
COLS = 128


@functools.lru_cache(maxsize=None)
def _gather_fn(BF, D, F, V1):
    ROWS = BF // COLS
    RW = ROWS // NW
    GB = 8
    NG = RW // GB
    NPW = RW * COLS

    mesh = plsc.VectorSubcoreMesh(core_axis_name="c", subcore_axis_name="s")

    @functools.partial(
        pl.kernel,
        out_type=jax.ShapeDtypeStruct((BF, D), jnp.float32),
        mesh=mesh,
        scratch_types=[
            pltpu.VMEM((RW, COLS), jnp.int32),
            pltpu.VMEM((GB * COLS, D), jnp.float32),
            pltpu.SemaphoreType.DMA,
        ],
        compiler_params=pltpu.CompilerParams(use_tc_tiling_on_sc=False),
    )
    def body(tab_ref, x_ref, out_ref, idx_v, rows_v, sem):
        wid = lax.axis_index("s") * NC + lax.axis_index("c")
        pltpu.sync_copy(x_ref.at[pl.ds(wid * RW, RW)], idx_v)

        @pl.loop(0, RW)
        def _row(r):
            for sub in range(COLS // LANES):
                c0 = sub * LANES
                p = r * COLS + c0 + lax.iota(jnp.int32, LANES)
                f = lax.rem(p, F)
                idx_v[r, pl.ds(c0, LANES)] = f * V1 + idx_v[r, pl.ds(c0, LANES)]

        out0 = wid * NPW

        @pl.loop(0, NG)
        def _grp(g):
            cps = [
                pltpu.async_copy(
                    tab_ref.at[idx_v.at[g * GB + b]],
                    rows_v.at[pl.ds(b * COLS, COLS)],
                    sem,
                )
                for b in range(GB)
            ]
            for cp in cps:
                cp.wait()
            pltpu.sync_copy(
                rows_v, out_ref.at[pl.ds(out0 + g * GB * COLS, GB * COLS)]
            )

    return body


def _mask_body(x_ref, o_ref):
    o_ref[...] = x_ref[...] == 0


def kernel(x, tables):
    F, V1, D = tables.shape
    B = x.shape[0]
    BF = B * F
    tab2 = tables.reshape(F * V1, D)
    x2 = x.reshape(BF // COLS, COLS)
    emb = _gather_fn(BF, D, F, V1)(tab2, x2)
    mask = pl.pallas_call(
        _mask_body,
        out_shape=jax.ShapeDtypeStruct((B, F), jnp.bool_),
    )(x)
    return emb.reshape(B, F, D), mask


# speedup vs baseline: 1.0004x; 1.0004x over previous
import functools

import jax
import jax.numpy as jnp
from jax import lax
from jax.experimental import pallas as pl
from jax.experimental.pallas import tpu as pltpu
from jax.experimental.pallas import tpu_sc as plsc

NC = 2
NS = 16
NW = NC * NS
LANES = 16
COLS = 128


@functools.lru_cache(maxsize=None)
def _gather_fn(BF, D, F, V1):
    NPW = BF // NW
    NG = 8
    G = NPW // NG

    mesh = plsc.VectorSubcoreMesh(core_axis_name="c", subcore_axis_name="s")

    @functools.partial(
        pl.kernel,
        out_type=jax.ShapeDtypeStruct((BF, D), jnp.float32),
        mesh=mesh,
        scratch_types=[
            pltpu.VMEM((NPW,), jnp.int32),
            pltpu.VMEM((G, D), jnp.float32),
            pltpu.SemaphoreType.DMA,
        ],
        compiler_params=pltpu.CompilerParams(use_tc_tiling_on_sc=False),
    )
    def body(tab_ref, x_ref, out_ref, idx_v, rows_v, sem):
        wid = lax.axis_index("s") * NC + lax.axis_index("c")
        pltpu.sync_copy(x_ref.at[pl.ds(wid * NPW, NPW)], idx_v)

        @pl.loop(0, NPW // LANES, unroll=8)
        def _slice(j):
            c0 = j * LANES
            p = c0 + lax.iota(jnp.int32, LANES)
            f = lax.rem(p, F)
            idx_v[pl.ds(c0, LANES)] = f * V1 + idx_v[pl.ds(c0, LANES)]

        out0 = wid * NPW

        @pl.loop(0, NG)
        def _grp(g):
            pltpu.async_copy(
                tab_ref.at[idx_v.at[pl.ds(g * G, G)]], rows_v, sem
            ).wait()
            pltpu.sync_copy(rows_v, out_ref.at[pl.ds(out0 + g * G, G)])

    return body


def _mask_body(x_ref, o_ref):
    o_ref[...] = x_ref[...] == 0


def kernel(x, tables):
    F, V1, D = tables.shape
    B = x.shape[0]
    BF = B * F
    tab2 = tables.reshape(F * V1, D)
    x2 = x.reshape(BF)
    emb = _gather_fn(BF, D, F, V1)(tab2, x2)
    mask = pl.pallas_call(
        _mask_body,
        out_shape=jax.ShapeDtypeStruct((B, F), jnp.bool_),
    )(x)
    return emb.reshape(B, F, D), mask


# speedup vs baseline: 29.6487x; 29.6371x over previous
import functools

import jax
import jax.numpy as jnp
from jax import lax
from jax.experimental import pallas as pl
from jax.experimental.pallas import tpu as pltpu
from jax.experimental.pallas import tpu_sc as plsc

NC = 2
NS = 16
NW = NC * NS
LANES = 16
CHUNK = 8192


@functools.lru_cache(maxsize=None)
def _lookup_fn(R, V1, B, D):
    RPW = R // NW
    mesh = plsc.VectorSubcoreMesh(core_axis_name="c", subcore_axis_name="s")

    @functools.partial(
        pl.kernel,
        out_type=jax.ShapeDtypeStruct((R, B), jnp.float32),
        mesh=mesh,
        scratch_types=[
            pltpu.VMEM((V1,), jnp.float32),
            pltpu.VMEM((B,), jnp.int32),
            pltpu.VMEM((CHUNK,), jnp.float32),
            pltpu.SemaphoreType.DMA,
        ],
        compiler_params=pltpu.CompilerParams(needs_layout_passes=False),
    )
    def body(tabT_ref, xT_ref, out_ref, row_v, xcol_v, oc_v, sem):
        w = lax.axis_index("s") * NC + lax.axis_index("c")
        r0 = w * RPW

        @pl.loop(0, RPW)
        def _row(i):
            r = r0 + i
            f = r // D
            pltpu.sync_copy(xT_ref.at[f], xcol_v)
            pltpu.sync_copy(tabT_ref.at[r], row_v)

            @pl.loop(0, B // CHUNK)
            def _chunk(c):
                @pl.loop(0, CHUNK // LANES, unroll=16)
                def _grp(k):
                    b0 = k * LANES
                    idx = xcol_v[pl.ds(c * CHUNK + b0, LANES)]
                    oc_v[pl.ds(b0, LANES)] = plsc.load_gather(row_v, [idx])

                pltpu.sync_copy(oc_v, out_ref.at[r, pl.ds(c * CHUNK, CHUNK)])

    return body


def _mask_body(x_ref, o_ref):
    o_ref[...] = x_ref[...] == 0


def kernel(x, tables):
    F, V1, D = tables.shape
    B = x.shape[0]
    tabT = tables.transpose(0, 2, 1).reshape(F * D, V1)
    xT = x.T
    outT = _lookup_fn(F * D, V1, B, D)(tabT, xT)
    emb = outT.reshape(F, D, B).transpose(2, 0, 1)
    maskT = pl.pallas_call(
        _mask_body,
        out_shape=jax.ShapeDtypeStruct((F, B), jnp.bool_),
    )(xT)
    return emb, maskT.T


# speedup vs baseline: 33.1829x; 1.1192x over previous
import functools

import jax
import jax.numpy as jnp
from jax import lax
from jax.experimental import pallas as pl
from jax.experimental.pallas import tpu as pltpu
from jax.experimental.pallas import tpu_sc as plsc

NC = 2
NS = 16
NW = NC * NS
LANES = 16
CHUNK = 2048
NBUF = 4


@functools.lru_cache(maxsize=None)
def _lookup_fn(R, V1, B, D):
    RPW = R // NW
    mesh = plsc.VectorSubcoreMesh(core_axis_name="c", subcore_axis_name="s")

    @functools.partial(
        pl.kernel,
        out_type=jax.ShapeDtypeStruct((R, B), jnp.float32),
        mesh=mesh,
        scratch_types=[
            pltpu.VMEM((V1,), jnp.float32),
            pltpu.VMEM((B,), jnp.int32),
            pltpu.VMEM((NBUF, CHUNK), jnp.float32),
            pltpu.SemaphoreType.DMA,
            pltpu.SemaphoreType.DMA,
        ],
        compiler_params=pltpu.CompilerParams(needs_layout_passes=False),
    )
    def body(tabT_ref, xT_ref, out_ref, row_v, xcol_v, oc_v, sem, sem_o):
        w = lax.axis_index("s") * NC + lax.axis_index("c")
        r0 = w * RPW
        rend = r0 + RPW
        f0 = r0 // D
        split = jnp.minimum((f0 + 1) * D, rend)

        def drain_one(r):
            pltpu.make_async_copy(
                oc_v.at[0], out_ref.at[r, pl.ds(0, CHUNK)], sem_o
            ).wait()

        def do_rows(f, lo, hi):
            pltpu.sync_copy(xT_ref.at[f], xcol_v)

            @pl.loop(lo, hi)
            def _row(r):
                pltpu.async_copy(tabT_ref.at[r], row_v, sem).wait()
                for c in range(B // CHUNK):
                    buf = c % NBUF

                    if c < NBUF:
                        @pl.when(r > r0)
                        def _():
                            drain_one(r)
                    else:
                        drain_one(r)

                    @pl.loop(0, CHUNK // LANES, unroll=16)
                    def _grp(k):
                        b0 = k * LANES
                        idx = xcol_v[pl.ds(c * CHUNK + b0, LANES)]
                        oc_v[buf, pl.ds(b0, LANES)] = plsc.load_gather(
                            row_v, [idx]
                        )

                    pltpu.async_copy(
                        oc_v.at[buf],
                        out_ref.at[r, pl.ds(c * CHUNK, CHUNK)],
                        sem_o,
                    )

        do_rows(f0, r0, split)

        @pl.when(split < rend)
        def _():
            do_rows(f0 + 1, split, rend)

        for _ in range(NBUF):
            drain_one(r0)

    return body


def _mask_body(x_ref, o_ref):
    o_ref[...] = x_ref[...] == 0


def kernel(x, tables):
    F, V1, D = tables.shape
    B = x.shape[0]
    tabT = tables.transpose(0, 2, 1).reshape(F * D, V1)
    xT = x.T
    outT = _lookup_fn(F * D, V1, B, D)(tabT, xT)
    emb = outT.reshape(F, D, B).transpose(2, 0, 1)
    maskT = pl.pallas_call(
        _mask_body,
        out_shape=jax.ShapeDtypeStruct((F, B), jnp.bool_),
    )(xT)
    return emb, maskT.T


# speedup vs baseline: 66.5443x; 2.0054x over previous
import functools

import jax
import jax.numpy as jnp
from jax import lax
from jax.experimental import pallas as pl
from jax.experimental.pallas import tpu as pltpu
from jax.experimental.pallas import tpu_sc as plsc

NC = 2
NS = 16
NW = NC * NS
LANES = 16
CHUNK = 2048
NBUF = 4


@functools.lru_cache(maxsize=None)
def _lookup_fn(R, V1, B, D):
    RPW = R // NW
    mesh = plsc.VectorSubcoreMesh(core_axis_name="c", subcore_axis_name="s")

    @functools.partial(
        pl.kernel,
        out_type=jax.ShapeDtypeStruct((R, B), jnp.float32),
        mesh=mesh,
        scratch_types=[
            pltpu.VMEM((V1,), jnp.float32),
            pltpu.VMEM((B,), jnp.int32),
            pltpu.VMEM((NBUF, CHUNK), jnp.float32),
            pltpu.SemaphoreType.DMA,
            pltpu.SemaphoreType.DMA,
        ],
        compiler_params=pltpu.CompilerParams(needs_layout_passes=False),
    )
    def body(tabT_ref, xT_ref, out_ref, row_v, xcol_v, oc_v, sem, sem_o):
        w = lax.axis_index("s") * NC + lax.axis_index("c")
        r0 = w * RPW
        rend = r0 + RPW
        f0 = r0 // D
        split = jnp.minimum((f0 + 1) * D, rend)

        def drain_one(r):
            pltpu.make_async_copy(
                oc_v.at[0], out_ref.at[r, pl.ds(0, CHUNK)], sem_o
            ).wait()

        def do_rows(f, lo, hi):
            pltpu.sync_copy(xT_ref.at[f], xcol_v)

            @pl.loop(lo, hi)
            def _row(r):
                pltpu.async_copy(tabT_ref.at[r], row_v, sem).wait()
                for c in range(B // CHUNK):
                    buf = c % NBUF

                    if c < NBUF:
                        @pl.when(r > r0)
                        def _():
                            drain_one(r)
                    else:
                        drain_one(r)

                    @plsc.parallel_loop(0, CHUNK // LANES, unroll=16)
                    def _grp(k):
                        b0 = k * LANES
                        idx = xcol_v[pl.ds(c * CHUNK + b0, LANES)]
                        oc_v[buf, pl.ds(b0, LANES)] = plsc.load_gather(
                            row_v, [idx]
                        )

                    pltpu.async_copy(
                        oc_v.at[buf],
                        out_ref.at[r, pl.ds(c * CHUNK, CHUNK)],
                        sem_o,
                    )

        do_rows(f0, r0, split)

        @pl.when(split < rend)
        def _():
            do_rows(f0 + 1, split, rend)

        for _ in range(NBUF):
            drain_one(r0)

    return body


def _mask_body(x_ref, o_ref):
    o_ref[...] = x_ref[...] == 0


def kernel(x, tables):
    F, V1, D = tables.shape
    B = x.shape[0]
    tabT = tables.transpose(0, 2, 1).reshape(F * D, V1)
    xT = x.T
    outT = _lookup_fn(F * D, V1, B, D)(tabT, xT)
    emb = outT.reshape(F, D, B).transpose(2, 0, 1)
    maskT = pl.pallas_call(
        _mask_body,
        out_shape=jax.ShapeDtypeStruct((F, B), jnp.bool_),
    )(xT)
    return emb, maskT.T
